# baseline (speedup 1.0000x reference)
import functools

import jax
import jax.numpy as jnp
from jax.experimental import pallas as pl
from jax.experimental.pallas import tpu as pltpu

_K = 64
_H = 16
_LN_EPS = 1e-5


def _ln(xx, g, b):
    mu = jnp.mean(xx, axis=-1, keepdims=True)
    var = jnp.mean((xx - mu) ** 2, axis=-1, keepdims=True)
    return (xx - mu) / jnp.sqrt(var + _LN_EPS) * g + b


def _softmax_last(s):
    s = s - jnp.max(s, axis=-1, keepdims=True)
    e = jnp.exp(s)
    return e / jnp.sum(e, axis=-1, keepdims=True)


def _stageA_body(x_ref, wfam_ref, femb_ref, wenc_ref, benc_ref,
                 fs_ref, basis_ref, mc_ref, sp_ref, *, nsteps, denom, n_iter):
    xx = x_ref[...]
    s = jnp.dot(xx, wfam_ref[...])
    s = s - jnp.max(s, axis=-1, keepdims=True)
    e = jnp.exp(s)
    p = e / jnp.sum(e, axis=-1, keepdims=True)
    fs_ref[...] = p
    basis = jnp.dot(p, femb_ref[...])
    basis_ref[...] = basis
    c = jnp.dot(xx - basis, wenc_ref[...]) + benc_ref[...]

    mu = jnp.mean(c, axis=1, keepdims=True)
    sg = jnp.sqrt(jnp.maximum(
        jnp.mean(c * c, axis=1, keepdims=True) - mu * mu, 0.0))
    lo0 = mu + sg * 1.7
    hi0 = mu + sg * 2.6

    def _cnt(t):
        return jnp.sum((c >= t).astype(jnp.float32), axis=1, keepdims=True)

    lo = jnp.where(_cnt(lo0) >= _K, lo0, mu - sg * 9.0)
    hi = jnp.where(_cnt(hi0) < _K, hi0, mu + sg * 9.0)

    def it(_, lh):
        lo_, hi_ = lh
        mid = (lo_ + hi_) * 0.5
        ge = _cnt(mid) >= _K
        return (jnp.where(ge, mid, lo_), jnp.where(ge, hi_, mid))

    lo, hi = jax.lax.fori_loop(0, n_iter, it, (lo, hi))
    masked = jnp.where(c >= lo, c, 0.0)
    mc_ref[...] = masked

    i = pl.program_id(0)

    @pl.when(i == 0)
    def _():
        sp_ref[...] = jnp.zeros((1, 1), jnp.float32)

    sp_ref[...] += jnp.sum(jnp.abs(masked)).reshape(1, 1)

    @pl.when(i == nsteps - 1)
    def _():
        sp_ref[...] = sp_ref[...] / denom


def _stageB_body(mc_ref, d_ref, basis_ref, bias_ref, g_ref, b_ref,
                 wq_ref, wk_ref, wv_ref,
                 q_ref, k_ref, v_ref, e_ref, *, nsteps, denom):
    i = pl.program_id(0)
    xr = (basis_ref[...] + bias_ref[...]
          + jnp.dot(mc_ref[...].astype(jnp.bfloat16), d_ref[...],
                    preferred_element_type=jnp.float32))

    @pl.when(i == 0)
    def _():
        e_ref[...] = jnp.zeros((1, 1), jnp.float32)

    e_ref[...] += jnp.sum(xr * xr).reshape(1, 1)

    @pl.when(i == nsteps - 1)
    def _():
        e_ref[...] = e_ref[...] / denom

    n = _ln(xr, g_ref[...], b_ref[...])
    q_ref[...] = jnp.dot(n, wq_ref[...])
    k_ref[...] = jnp.dot(n, wk_ref[...])
    v_ref[...] = jnp.dot(n, wv_ref[...])


def _attn_body(q_ref, k_ref, v_ref, w_ref, ctx_ref, kst_ref, vst_ref,
               *, scale, dh, t_len):
    i = pl.program_id(2)

    @pl.when(i == 0)
    def _():
        k2 = k_ref[0]
        v2 = v_ref[0]
        z = jnp.zeros((t_len, dh), jnp.float32)
        kst_ref[...] = jnp.concatenate([
            jnp.concatenate([k2[:, :dh], z], axis=1),
            jnp.concatenate([z, k2[:, dh:]], axis=1)], axis=0)
        vst_ref[...] = jnp.concatenate([
            jnp.concatenate([v2[:, :dh], z], axis=1),
            jnp.concatenate([z, v2[:, dh:]], axis=1)], axis=0)

    q2 = q_ref[0]
    s_cat = jax.lax.dot_general(q2, kst_ref[...], (((1,), (1,)), ((), ()))) * scale
    w0 = _softmax_last(s_cat[:, :t_len])
    w1 = _softmax_last(s_cat[:, t_len:])
    w_ref[0, 0] = w0
    w_ref[0, 1] = w1
    w_cat = jnp.concatenate([w0, w1], axis=1)
    ctx_ref[0] = jnp.dot(w_cat, vst_ref[...])


def _stageF_body(c_ref, wo_ref, x_ref, g_ref, b_ref, wg_ref, w1_ref, w2_ref,
                 xo_ref, e_ref, *, nsteps, denom):
    f32 = jnp.float32
    bf16 = jnp.bfloat16
    x1 = jnp.dot(c_ref[...].astype(bf16), wo_ref[...],
                 preferred_element_type=f32) + x_ref[...]
    i = pl.program_id(0)

    @pl.when(i == 0)
    def _():
        e_ref[...] = jnp.zeros((1, 1), f32)

    e_ref[...] += jnp.sum(x1 * x1).reshape(1, 1)

    @pl.when(i == nsteps - 1)
    def _():
        e_ref[...] = e_ref[...] / denom

    nb = _ln(x1, g_ref[...], b_ref[...]).astype(bf16)
    h = (jax.nn.sigmoid(jnp.dot(nb, wg_ref[...], preferred_element_type=f32))
         * jax.nn.gelu(jnp.dot(nb, w1_ref[...], preferred_element_type=f32)))
    xo_ref[...] = x1 + jnp.dot(h.astype(bf16), w2_ref[...],
                               preferred_element_type=f32)


def kernel(x, W_fam, fam_emb, W_enc, b_enc, dictionary, bias_correction,
           ln1_g, ln1_b, Wq, Wk, Wv, Wo, ln2_g, ln2_b, W1, Wg, W2):
    B, T, D = x.shape
    F = W_fam.shape[1]
    M = W_enc.shape[1]
    DFF = W1.shape[1]
    N = B * T
    H = _H
    dh = D // H

    R = 256
    nt = N // R
    TQ = 256
    f32 = jnp.float32
    bf16 = jnp.bfloat16

    xf = x.reshape(N, D)

    fs, basis, masked, sp = pl.pallas_call(
        functools.partial(_stageA_body, nsteps=nt, denom=float(N * M), n_iter=18),
        grid=(nt,),
        in_specs=[
            pl.BlockSpec((R, D), lambda i: (i, 0)),
            pl.BlockSpec((D, F), lambda i: (0, 0)),
            pl.BlockSpec((F, D), lambda i: (0, 0)),
            pl.BlockSpec((D, M), lambda i: (0, 0)),
            pl.BlockSpec((1, M), lambda i: (0, 0)),
        ],
        out_specs=[
            pl.BlockSpec((R, F), lambda i: (i, 0)),
            pl.BlockSpec((R, D), lambda i: (i, 0)),
            pl.BlockSpec((R, M), lambda i: (i, 0)),
            pl.BlockSpec((1, 1), lambda i: (0, 0)),
        ],
        out_shape=[
            jax.ShapeDtypeStruct((N, F), f32),
            jax.ShapeDtypeStruct((N, D), f32),
            jax.ShapeDtypeStruct((N, M), f32),
            jax.ShapeDtypeStruct((1, 1), f32),
        ],
    )(xf, W_fam, fam_emb, W_enc, b_enc.reshape(1, M))

    qf, kf, vf, e1 = pl.pallas_call(
        functools.partial(_stageB_body, nsteps=nt, denom=float(N * D)),
        grid=(nt,),
        in_specs=[
            pl.BlockSpec((R, M), lambda i: (i, 0)),
            pl.BlockSpec((M, D), lambda i: (0, 0)),
            pl.BlockSpec((R, D), lambda i: (i, 0)),
            pl.BlockSpec((1, D), lambda i: (0, 0)),
            pl.BlockSpec((1, D), lambda i: (0, 0)),
            pl.BlockSpec((1, D), lambda i: (0, 0)),
            pl.BlockSpec((D, D), lambda i: (0, 0)),
            pl.BlockSpec((D, D), lambda i: (0, 0)),
            pl.BlockSpec((D, D), lambda i: (0, 0)),
        ],
        out_specs=[
            pl.BlockSpec((R, D), lambda i: (i, 0)),
            pl.BlockSpec((R, D), lambda i: (i, 0)),
            pl.BlockSpec((R, D), lambda i: (i, 0)),
            pl.BlockSpec((1, 1), lambda i: (0, 0)),
        ],
        out_shape=[
            jax.ShapeDtypeStruct((N, D), f32),
            jax.ShapeDtypeStruct((N, D), f32),
            jax.ShapeDtypeStruct((N, D), f32),
            jax.ShapeDtypeStruct((1, 1), f32),
        ],
    )(masked, dictionary.astype(bf16), basis, bias_correction.reshape(1, D),
      ln1_g.reshape(1, D), ln1_b.reshape(1, D), Wq, Wk, Wv)

    q3 = qf.reshape(B, T, D)
    k3 = kf.reshape(B, T, D)
    v3 = vf.reshape(B, T, D)

    attn_w, ctx3 = pl.pallas_call(
        functools.partial(_attn_body, scale=1.0 / float(dh) ** 0.5, dh=dh, t_len=T),
        grid=(B, H // 2, T // TQ),
        in_specs=[
            pl.BlockSpec((1, TQ, 2 * dh), lambda b, hp, i: (b, i, hp)),
            pl.BlockSpec((1, T, 2 * dh), lambda b, hp, i: (b, 0, hp)),
            pl.BlockSpec((1, T, 2 * dh), lambda b, hp, i: (b, 0, hp)),
        ],
        out_specs=[
            pl.BlockSpec((1, 2, TQ, T), lambda b, hp, i: (b, hp, i, 0)),
            pl.BlockSpec((1, TQ, 2 * dh), lambda b, hp, i: (b, i, hp)),
        ],
        out_shape=[
            jax.ShapeDtypeStruct((B, H, T, T), f32),
            jax.ShapeDtypeStruct((B, T, D), f32),
        ],
        scratch_shapes=[
            pltpu.VMEM((2 * T, 2 * dh), f32),
            pltpu.VMEM((2 * T, 2 * dh), f32),
        ],
    )(q3, k3, v3)

    ctx = ctx3.reshape(N, D)

    xout, e2 = pl.pallas_call(
        functools.partial(_stageF_body, nsteps=nt, denom=float(N * D)),
        grid=(nt,),
        in_specs=[
            pl.BlockSpec((R, D), lambda i: (i, 0)),
            pl.BlockSpec((D, D), lambda i: (0, 0)),
            pl.BlockSpec((R, D), lambda i: (i, 0)),
            pl.BlockSpec((1, D), lambda i: (0, 0)),
            pl.BlockSpec((1, D), lambda i: (0, 0)),
            pl.BlockSpec((D, DFF), lambda i: (0, 0)),
            pl.BlockSpec((D, DFF), lambda i: (0, 0)),
            pl.BlockSpec((DFF, D), lambda i: (0, 0)),
        ],
        out_specs=[
            pl.BlockSpec((R, D), lambda i: (i, 0)),
            pl.BlockSpec((1, 1), lambda i: (0, 0)),
        ],
        out_shape=[
            jax.ShapeDtypeStruct((N, D), f32),
            jax.ShapeDtypeStruct((1, 1), f32),
        ],
    )(ctx, Wo.astype(bf16), xf, ln2_g.reshape(1, D), ln2_b.reshape(1, D),
      Wg.astype(bf16), W1.astype(bf16), W2.astype(bf16))

    return (
        xout.reshape(B, T, D),
        attn_w,
        fs.reshape(B, T, F),
        sp.reshape(()),
        e1.reshape(()),
        e2.reshape(()),
    )


# speedup vs baseline: 1.0309x; 1.0309x over previous
import functools

import jax
import jax.numpy as jnp
from jax.experimental import pallas as pl
from jax.experimental.pallas import tpu as pltpu

_K = 64
_H = 16
_LN_EPS = 1e-5


def _ln(xx, g, b):
    mu = jnp.mean(xx, axis=-1, keepdims=True)
    var = jnp.mean((xx - mu) ** 2, axis=-1, keepdims=True)
    return (xx - mu) / jnp.sqrt(var + _LN_EPS) * g + b


def _softmax_last(s):
    s = s - jnp.max(s, axis=-1, keepdims=True)
    e = jnp.exp(s)
    return e / jnp.sum(e, axis=-1, keepdims=True)


def _stageA_body(x_ref, wfam_ref, femb_ref, wenc_ref, benc_ref,
                 fs_ref, basis_ref, mc_ref, sp_ref, *, nsteps, denom, n_iter):
    xx = x_ref[...]
    s = jnp.dot(xx, wfam_ref[...])
    s = s - jnp.max(s, axis=-1, keepdims=True)
    e = jnp.exp(s)
    p = e / jnp.sum(e, axis=-1, keepdims=True)
    fs_ref[...] = p
    basis = jnp.dot(p, femb_ref[...])
    basis_ref[...] = basis
    c = jnp.dot(xx - basis, wenc_ref[...]) + benc_ref[...]

    mu = jnp.mean(c, axis=1, keepdims=True)
    sg = jnp.sqrt(jnp.maximum(
        jnp.mean(c * c, axis=1, keepdims=True) - mu * mu, 0.0))
    lo0 = mu + sg * 1.7
    hi0 = mu + sg * 2.6

    def _cnt(t):
        return jnp.sum((c >= t).astype(jnp.float32), axis=1, keepdims=True)

    lo = jnp.where(_cnt(lo0) >= _K, lo0, mu - sg * 9.0)
    hi = jnp.where(_cnt(hi0) < _K, hi0, mu + sg * 9.0)

    def it(_, lh):
        lo_, hi_ = lh
        mid = (lo_ + hi_) * 0.5
        ge = _cnt(mid) >= _K
        return (jnp.where(ge, mid, lo_), jnp.where(ge, hi_, mid))

    lo, hi = jax.lax.fori_loop(0, n_iter, it, (lo, hi))
    masked = jnp.where(c >= lo, c, 0.0)
    mc_ref[...] = masked

    i = pl.program_id(0)

    @pl.when(i == 0)
    def _():
        sp_ref[...] = jnp.zeros((1, 1), jnp.float32)

    sp_ref[...] += jnp.sum(jnp.abs(masked)).reshape(1, 1)

    @pl.when(i == nsteps - 1)
    def _():
        sp_ref[...] = sp_ref[...] / denom


def _stageB_body(mc_ref, d_ref, basis_ref, bias_ref, g_ref, b_ref,
                 wq_ref, wk_ref, wv_ref,
                 q_ref, k_ref, v_ref, e_ref, *, nsteps, denom):
    i = pl.program_id(0)
    xr = (basis_ref[...] + bias_ref[...]
          + jnp.dot(mc_ref[...], d_ref[...]))

    @pl.when(i == 0)
    def _():
        e_ref[...] = jnp.zeros((1, 1), jnp.float32)

    e_ref[...] += jnp.sum(xr * xr).reshape(1, 1)

    @pl.when(i == nsteps - 1)
    def _():
        e_ref[...] = e_ref[...] / denom

    n = _ln(xr, g_ref[...], b_ref[...])
    q_ref[...] = jnp.dot(n, wq_ref[...])
    k_ref[...] = jnp.dot(n, wk_ref[...])
    v_ref[...] = jnp.dot(n, wv_ref[...])


def _attn_body(q_ref, k_ref, v_ref, w_ref, ctx_ref, kst_ref, vst_ref,
               *, scale, dh, t_len):
    i = pl.program_id(2)

    @pl.when(i == 0)
    def _():
        k2 = k_ref[0]
        v2 = v_ref[0]
        z = jnp.zeros((t_len, dh), jnp.float32)
        kst_ref[...] = jnp.concatenate([
            jnp.concatenate([k2[:, :dh], z], axis=1),
            jnp.concatenate([z, k2[:, dh:]], axis=1)], axis=0)
        vst_ref[...] = jnp.concatenate([
            jnp.concatenate([v2[:, :dh], z], axis=1),
            jnp.concatenate([z, v2[:, dh:]], axis=1)], axis=0)

    q2 = q_ref[0]
    s_cat = jax.lax.dot_general(q2, kst_ref[...], (((1,), (1,)), ((), ()))) * scale
    w0 = _softmax_last(s_cat[:, :t_len])
    w1 = _softmax_last(s_cat[:, t_len:])
    w_ref[0, 0] = w0
    w_ref[0, 1] = w1
    w_cat = jnp.concatenate([w0, w1], axis=1)
    ctx_ref[0] = jnp.dot(w_cat, vst_ref[...])


def _stageF_body(c_ref, wo_ref, x_ref, g_ref, b_ref, wg_ref, w1_ref, w2_ref,
                 xo_ref, e_ref, *, nsteps, denom):
    f32 = jnp.float32
    bf16 = jnp.bfloat16
    x1 = jnp.dot(c_ref[...].astype(bf16), wo_ref[...],
                 preferred_element_type=f32) + x_ref[...]
    i = pl.program_id(0)

    @pl.when(i == 0)
    def _():
        e_ref[...] = jnp.zeros((1, 1), f32)

    e_ref[...] += jnp.sum(x1 * x1).reshape(1, 1)

    @pl.when(i == nsteps - 1)
    def _():
        e_ref[...] = e_ref[...] / denom

    nb = _ln(x1, g_ref[...], b_ref[...]).astype(bf16)
    h = (jax.nn.sigmoid(jnp.dot(nb, wg_ref[...], preferred_element_type=f32))
         * jax.nn.gelu(jnp.dot(nb, w1_ref[...], preferred_element_type=f32)))
    xo_ref[...] = x1 + jnp.dot(h.astype(bf16), w2_ref[...],
                               preferred_element_type=f32)


def kernel(x, W_fam, fam_emb, W_enc, b_enc, dictionary, bias_correction,
           ln1_g, ln1_b, Wq, Wk, Wv, Wo, ln2_g, ln2_b, W1, Wg, W2):
    B, T, D = x.shape
    F = W_fam.shape[1]
    M = W_enc.shape[1]
    DFF = W1.shape[1]
    N = B * T
    H = _H
    dh = D // H

    R = 256
    nt = N // R
    TQ = 512
    f32 = jnp.float32
    bf16 = jnp.bfloat16

    xf = x.reshape(N, D)

    fs, basis, masked, sp = pl.pallas_call(
        functools.partial(_stageA_body, nsteps=nt, denom=float(N * M), n_iter=18),
        grid=(nt,),
        in_specs=[
            pl.BlockSpec((R, D), lambda i: (i, 0)),
            pl.BlockSpec((D, F), lambda i: (0, 0)),
            pl.BlockSpec((F, D), lambda i: (0, 0)),
            pl.BlockSpec((D, M), lambda i: (0, 0)),
            pl.BlockSpec((1, M), lambda i: (0, 0)),
        ],
        out_specs=[
            pl.BlockSpec((R, F), lambda i: (i, 0)),
            pl.BlockSpec((R, D), lambda i: (i, 0)),
            pl.BlockSpec((R, M), lambda i: (i, 0)),
            pl.BlockSpec((1, 1), lambda i: (0, 0)),
        ],
        out_shape=[
            jax.ShapeDtypeStruct((N, F), f32),
            jax.ShapeDtypeStruct((N, D), f32),
            jax.ShapeDtypeStruct((N, M), f32),
            jax.ShapeDtypeStruct((1, 1), f32),
        ],
    )(xf, W_fam, fam_emb, W_enc, b_enc.reshape(1, M))

    qf, kf, vf, e1 = pl.pallas_call(
        functools.partial(_stageB_body, nsteps=nt, denom=float(N * D)),
        grid=(nt,),
        in_specs=[
            pl.BlockSpec((R, M), lambda i: (i, 0)),
            pl.BlockSpec((M, D), lambda i: (0, 0)),
            pl.BlockSpec((R, D), lambda i: (i, 0)),
            pl.BlockSpec((1, D), lambda i: (0, 0)),
            pl.BlockSpec((1, D), lambda i: (0, 0)),
            pl.BlockSpec((1, D), lambda i: (0, 0)),
            pl.BlockSpec((D, D), lambda i: (0, 0)),
            pl.BlockSpec((D, D), lambda i: (0, 0)),
            pl.BlockSpec((D, D), lambda i: (0, 0)),
        ],
        out_specs=[
            pl.BlockSpec((R, D), lambda i: (i, 0)),
            pl.BlockSpec((R, D), lambda i: (i, 0)),
            pl.BlockSpec((R, D), lambda i: (i, 0)),
            pl.BlockSpec((1, 1), lambda i: (0, 0)),
        ],
        out_shape=[
            jax.ShapeDtypeStruct((N, D), f32),
            jax.ShapeDtypeStruct((N, D), f32),
            jax.ShapeDtypeStruct((N, D), f32),
            jax.ShapeDtypeStruct((1, 1), f32),
        ],
    )(masked, dictionary, basis, bias_correction.reshape(1, D),
      ln1_g.reshape(1, D), ln1_b.reshape(1, D), Wq, Wk, Wv)

    q3 = qf.reshape(B, T, D)
    k3 = kf.reshape(B, T, D)
    v3 = vf.reshape(B, T, D)

    attn_w, ctx3 = pl.pallas_call(
        functools.partial(_attn_body, scale=1.0 / float(dh) ** 0.5, dh=dh, t_len=T),
        grid=(B, H // 2, T // TQ),
        in_specs=[
            pl.BlockSpec((1, TQ, 2 * dh), lambda b, hp, i: (b, i, hp)),
            pl.BlockSpec((1, T, 2 * dh), lambda b, hp, i: (b, 0, hp)),
            pl.BlockSpec((1, T, 2 * dh), lambda b, hp, i: (b, 0, hp)),
        ],
        out_specs=[
            pl.BlockSpec((1, 2, TQ, T), lambda b, hp, i: (b, hp, i, 0)),
            pl.BlockSpec((1, TQ, 2 * dh), lambda b, hp, i: (b, i, hp)),
        ],
        out_shape=[
            jax.ShapeDtypeStruct((B, H, T, T), f32),
            jax.ShapeDtypeStruct((B, T, D), f32),
        ],
        scratch_shapes=[
            pltpu.VMEM((2 * T, 2 * dh), f32),
            pltpu.VMEM((2 * T, 2 * dh), f32),
        ],
    )(q3, k3, v3)

    ctx = ctx3.reshape(N, D)

    xout, e2 = pl.pallas_call(
        functools.partial(_stageF_body, nsteps=nt, denom=float(N * D)),
        grid=(nt,),
        in_specs=[
            pl.BlockSpec((R, D), lambda i: (i, 0)),
            pl.BlockSpec((D, D), lambda i: (0, 0)),
            pl.BlockSpec((R, D), lambda i: (i, 0)),
            pl.BlockSpec((1, D), lambda i: (0, 0)),
            pl.BlockSpec((1, D), lambda i: (0, 0)),
            pl.BlockSpec((D, DFF), lambda i: (0, 0)),
            pl.BlockSpec((D, DFF), lambda i: (0, 0)),
            pl.BlockSpec((DFF, D), lambda i: (0, 0)),
        ],
        out_specs=[
            pl.BlockSpec((R, D), lambda i: (i, 0)),
            pl.BlockSpec((1, 1), lambda i: (0, 0)),
        ],
        out_shape=[
            jax.ShapeDtypeStruct((N, D), f32),
            jax.ShapeDtypeStruct((1, 1), f32),
        ],
    )(ctx, Wo.astype(bf16), xf, ln2_g.reshape(1, D), ln2_b.reshape(1, D),
      Wg.astype(bf16), W1.astype(bf16), W2.astype(bf16))

    return (
        xout.reshape(B, T, D),
        attn_w,
        fs.reshape(B, T, F),
        sp.reshape(()),
        e1.reshape(()),
        e2.reshape(()),
    )


# speedup vs baseline: 1.0423x; 1.0110x over previous
import functools

import jax
import jax.numpy as jnp
from jax.experimental import pallas as pl
from jax.experimental.pallas import tpu as pltpu

_K = 64
_H = 16
_LN_EPS = 1e-5


def _ln(xx, g, b):
    mu = jnp.mean(xx, axis=-1, keepdims=True)
    var = jnp.mean((xx - mu) ** 2, axis=-1, keepdims=True)
    return (xx - mu) / jnp.sqrt(var + _LN_EPS) * g + b


def _softmax_last(s):
    s = s - jnp.max(s, axis=-1, keepdims=True)
    e = jnp.exp(s)
    return e / jnp.sum(e, axis=-1, keepdims=True)


def _stageA_body(x_ref, wfam_ref, femb_ref, wenc_ref, benc_ref,
                 fs_ref, basis_ref, mc_ref, sp_ref, *, nsteps, denom, n_iter):
    xx = x_ref[...]
    s = jnp.dot(xx, wfam_ref[...])
    s = s - jnp.max(s, axis=-1, keepdims=True)
    e = jnp.exp(s)
    p = e / jnp.sum(e, axis=-1, keepdims=True)
    fs_ref[...] = p
    basis = jnp.dot(p, femb_ref[...])
    basis_ref[...] = basis
    c = jnp.dot(xx - basis, wenc_ref[...]) + benc_ref[...]

    mu = jnp.mean(c, axis=1, keepdims=True)
    sg = jnp.sqrt(jnp.maximum(
        jnp.mean(c * c, axis=1, keepdims=True) - mu * mu, 0.0))
    lo0 = mu + sg * 1.7
    hi0 = mu + sg * 2.6

    def _cnt(t):
        return jnp.sum((c >= t).astype(jnp.float32), axis=1, keepdims=True)

    lo = jnp.where(_cnt(lo0) >= _K, lo0, mu - sg * 9.0)
    hi = jnp.where(_cnt(hi0) < _K, hi0, mu + sg * 9.0)

    def it(_, lh):
        lo_, hi_ = lh
        mid = (lo_ + hi_) * 0.5
        ge = _cnt(mid) >= _K
        return (jnp.where(ge, mid, lo_), jnp.where(ge, hi_, mid))

    lo, hi = jax.lax.fori_loop(0, n_iter, it, (lo, hi))
    masked = jnp.where(c >= lo, c, 0.0)
    mc_ref[...] = masked

    i = pl.program_id(0)

    @pl.when(i == 0)
    def _():
        sp_ref[...] = jnp.zeros((1, 1), jnp.float32)

    sp_ref[...] += jnp.sum(jnp.abs(masked)).reshape(1, 1)

    @pl.when(i == nsteps - 1)
    def _():
        sp_ref[...] = sp_ref[...] / denom


def _stageB_body(mc_ref, d_ref, basis_ref, bias_ref, g_ref, b_ref,
                 wq_ref, wk_ref, wv_ref,
                 q_ref, k_ref, v_ref, e_ref, *, nsteps, denom):
    i = pl.program_id(0)
    xr = (basis_ref[...] + bias_ref[...]
          + jnp.dot(mc_ref[...], d_ref[...]))

    @pl.when(i == 0)
    def _():
        e_ref[...] = jnp.zeros((1, 1), jnp.float32)

    e_ref[...] += jnp.sum(xr * xr).reshape(1, 1)

    @pl.when(i == nsteps - 1)
    def _():
        e_ref[...] = e_ref[...] / denom

    n = _ln(xr, g_ref[...], b_ref[...])
    q_ref[...] = jnp.dot(n, wq_ref[...])
    k_ref[...] = jnp.dot(n, wk_ref[...])
    v_ref[...] = jnp.dot(n, wv_ref[...])


def _attn_body(q_ref, k_ref, v_ref, w_ref, ctx_ref, kst_ref, vst_ref,
               *, scale, dh, t_len):
    i = pl.program_id(2)

    @pl.when(i == 0)
    def _():
        k2 = k_ref[0]
        v2 = v_ref[0]
        z = jnp.zeros((t_len, dh), jnp.float32)
        kst_ref[...] = jnp.concatenate([
            jnp.concatenate([k2[:, :dh], z], axis=1),
            jnp.concatenate([z, k2[:, dh:]], axis=1)], axis=0)
        vst_ref[...] = jnp.concatenate([
            jnp.concatenate([v2[:, :dh], z], axis=1),
            jnp.concatenate([z, v2[:, dh:]], axis=1)], axis=0)

    q2 = q_ref[0]
    s_cat = jax.lax.dot_general(q2, kst_ref[...], (((1,), (1,)), ((), ()))) * scale
    w0 = _softmax_last(s_cat[:, :t_len])
    w1 = _softmax_last(s_cat[:, t_len:])
    w_ref[0, 0] = w0
    w_ref[0, 1] = w1
    w_cat = jnp.concatenate([w0, w1], axis=1)
    ctx_ref[0] = jnp.dot(w_cat, vst_ref[...])


def _stageF_body(c_ref, wo_ref, x_ref, g_ref, b_ref, wg_ref, w1_ref, w2_ref,
                 xo_ref, e_ref, *, nsteps, denom):
    f32 = jnp.float32
    bf16 = jnp.bfloat16
    x1 = jnp.dot(c_ref[...].astype(bf16), wo_ref[...],
                 preferred_element_type=f32) + x_ref[...]
    i = pl.program_id(0)

    @pl.when(i == 0)
    def _():
        e_ref[...] = jnp.zeros((1, 1), f32)

    e_ref[...] += jnp.sum(x1 * x1).reshape(1, 1)

    @pl.when(i == nsteps - 1)
    def _():
        e_ref[...] = e_ref[...] / denom

    nb = _ln(x1, g_ref[...], b_ref[...]).astype(bf16)
    h = (jax.nn.sigmoid(jnp.dot(nb, wg_ref[...], preferred_element_type=f32))
         * jax.nn.gelu(jnp.dot(nb, w1_ref[...], preferred_element_type=f32)))
    xo_ref[...] = x1 + jnp.dot(h.astype(bf16), w2_ref[...],
                               preferred_element_type=f32)


def kernel(x, W_fam, fam_emb, W_enc, b_enc, dictionary, bias_correction,
           ln1_g, ln1_b, Wq, Wk, Wv, Wo, ln2_g, ln2_b, W1, Wg, W2):
    B, T, D = x.shape
    F = W_fam.shape[1]
    M = W_enc.shape[1]
    DFF = W1.shape[1]
    N = B * T
    H = _H
    dh = D // H

    R = 256
    nt = N // R
    TQ = 1024
    f32 = jnp.float32
    bf16 = jnp.bfloat16

    xf = x.reshape(N, D)

    fs, basis, masked, sp = pl.pallas_call(
        functools.partial(_stageA_body, nsteps=nt, denom=float(N * M), n_iter=18),
        grid=(nt,),
        in_specs=[
            pl.BlockSpec((R, D), lambda i: (i, 0)),
            pl.BlockSpec((D, F), lambda i: (0, 0)),
            pl.BlockSpec((F, D), lambda i: (0, 0)),
            pl.BlockSpec((D, M), lambda i: (0, 0)),
            pl.BlockSpec((1, M), lambda i: (0, 0)),
        ],
        out_specs=[
            pl.BlockSpec((R, F), lambda i: (i, 0)),
            pl.BlockSpec((R, D), lambda i: (i, 0)),
            pl.BlockSpec((R, M), lambda i: (i, 0)),
            pl.BlockSpec((1, 1), lambda i: (0, 0)),
        ],
        out_shape=[
            jax.ShapeDtypeStruct((N, F), f32),
            jax.ShapeDtypeStruct((N, D), f32),
            jax.ShapeDtypeStruct((N, M), f32),
            jax.ShapeDtypeStruct((1, 1), f32),
        ],
    )(xf, W_fam, fam_emb, W_enc, b_enc.reshape(1, M))

    qf, kf, vf, e1 = pl.pallas_call(
        functools.partial(_stageB_body, nsteps=nt, denom=float(N * D)),
        grid=(nt,),
        in_specs=[
            pl.BlockSpec((R, M), lambda i: (i, 0)),
            pl.BlockSpec((M, D), lambda i: (0, 0)),
            pl.BlockSpec((R, D), lambda i: (i, 0)),
            pl.BlockSpec((1, D), lambda i: (0, 0)),
            pl.BlockSpec((1, D), lambda i: (0, 0)),
            pl.BlockSpec((1, D), lambda i: (0, 0)),
            pl.BlockSpec((D, D), lambda i: (0, 0)),
            pl.BlockSpec((D, D), lambda i: (0, 0)),
            pl.BlockSpec((D, D), lambda i: (0, 0)),
        ],
        out_specs=[
            pl.BlockSpec((R, D), lambda i: (i, 0)),
            pl.BlockSpec((R, D), lambda i: (i, 0)),
            pl.BlockSpec((R, D), lambda i: (i, 0)),
            pl.BlockSpec((1, 1), lambda i: (0, 0)),
        ],
        out_shape=[
            jax.ShapeDtypeStruct((N, D), f32),
            jax.ShapeDtypeStruct((N, D), f32),
            jax.ShapeDtypeStruct((N, D), f32),
            jax.ShapeDtypeStruct((1, 1), f32),
        ],
    )(masked, dictionary, basis, bias_correction.reshape(1, D),
      ln1_g.reshape(1, D), ln1_b.reshape(1, D), Wq, Wk, Wv)

    q3 = qf.reshape(B, T, D)
    k3 = kf.reshape(B, T, D)
    v3 = vf.reshape(B, T, D)

    attn_w, ctx3 = pl.pallas_call(
        functools.partial(_attn_body, scale=1.0 / float(dh) ** 0.5, dh=dh, t_len=T),
        grid=(B, H // 2, T // TQ),
        in_specs=[
            pl.BlockSpec((1, TQ, 2 * dh), lambda b, hp, i: (b, i, hp)),
            pl.BlockSpec((1, T, 2 * dh), lambda b, hp, i: (b, 0, hp)),
            pl.BlockSpec((1, T, 2 * dh), lambda b, hp, i: (b, 0, hp)),
        ],
        out_specs=[
            pl.BlockSpec((1, 2, TQ, T), lambda b, hp, i: (b, hp, i, 0)),
            pl.BlockSpec((1, TQ, 2 * dh), lambda b, hp, i: (b, i, hp)),
        ],
        out_shape=[
            jax.ShapeDtypeStruct((B, H, T, T), f32),
            jax.ShapeDtypeStruct((B, T, D), f32),
        ],
        scratch_shapes=[
            pltpu.VMEM((2 * T, 2 * dh), f32),
            pltpu.VMEM((2 * T, 2 * dh), f32),
        ],
    )(q3, k3, v3)

    ctx = ctx3.reshape(N, D)

    xout, e2 = pl.pallas_call(
        functools.partial(_stageF_body, nsteps=nt, denom=float(N * D)),
        grid=(nt,),
        in_specs=[
            pl.BlockSpec((R, D), lambda i: (i, 0)),
            pl.BlockSpec((D, D), lambda i: (0, 0)),
            pl.BlockSpec((R, D), lambda i: (i, 0)),
            pl.BlockSpec((1, D), lambda i: (0, 0)),
            pl.BlockSpec((1, D), lambda i: (0, 0)),
            pl.BlockSpec((D, DFF), lambda i: (0, 0)),
            pl.BlockSpec((D, DFF), lambda i: (0, 0)),
            pl.BlockSpec((DFF, D), lambda i: (0, 0)),
        ],
        out_specs=[
            pl.BlockSpec((R, D), lambda i: (i, 0)),
            pl.BlockSpec((1, 1), lambda i: (0, 0)),
        ],
        out_shape=[
            jax.ShapeDtypeStruct((N, D), f32),
            jax.ShapeDtypeStruct((1, 1), f32),
        ],
    )(ctx, Wo.astype(bf16), xf, ln2_g.reshape(1, D), ln2_b.reshape(1, D),
      Wg.astype(bf16), W1.astype(bf16), W2.astype(bf16))

    return (
        xout.reshape(B, T, D),
        attn_w,
        fs.reshape(B, T, F),
        sp.reshape(()),
        e1.reshape(()),
        e2.reshape(()),
    )


# speedup vs baseline: 1.0570x; 1.0141x over previous
import functools

import jax
import jax.numpy as jnp
from jax.experimental import pallas as pl
from jax.experimental.pallas import tpu as pltpu

_K = 64
_H = 16
_LN_EPS = 1e-5


def _ln(xx, g, b):
    mu = jnp.mean(xx, axis=-1, keepdims=True)
    var = jnp.mean((xx - mu) ** 2, axis=-1, keepdims=True)
    return (xx - mu) / jnp.sqrt(var + _LN_EPS) * g + b


def _softmax_last(s):
    s = s - jnp.max(s, axis=-1, keepdims=True)
    e = jnp.exp(s)
    return e / jnp.sum(e, axis=-1, keepdims=True)


def _stageA_body(x_ref, wfam_ref, femb_ref, wenc_ref, benc_ref,
                 fs_ref, basis_ref, mc_ref, sp_ref, *, nsteps, denom, n_iter):
    xx = x_ref[...]
    s = jnp.dot(xx, wfam_ref[...])
    s = s - jnp.max(s, axis=-1, keepdims=True)
    e = jnp.exp(s)
    p = e / jnp.sum(e, axis=-1, keepdims=True)
    fs_ref[...] = p
    basis = jnp.dot(p, femb_ref[...])
    basis_ref[...] = basis
    c = jnp.dot(xx - basis, wenc_ref[...]) + benc_ref[...]

    mu = jnp.mean(c, axis=1, keepdims=True)
    sg = jnp.sqrt(jnp.maximum(
        jnp.mean(c * c, axis=1, keepdims=True) - mu * mu, 0.0))
    lo0 = mu + sg * 1.7
    hi0 = mu + sg * 2.6

    def _cnt(t):
        return jnp.sum((c >= t).astype(jnp.float32), axis=1, keepdims=True)

    lo = jnp.where(_cnt(lo0) >= _K, lo0, mu - sg * 9.0)
    hi = jnp.where(_cnt(hi0) < _K, hi0, mu + sg * 9.0)

    def it(_, lh):
        lo_, hi_ = lh
        mid = (lo_ + hi_) * 0.5
        ge = _cnt(mid) >= _K
        return (jnp.where(ge, mid, lo_), jnp.where(ge, hi_, mid))

    lo, hi = jax.lax.fori_loop(0, n_iter, it, (lo, hi))
    masked = jnp.where(c >= lo, c, 0.0)
    mc_ref[...] = masked

    i = pl.program_id(0)

    @pl.when(i == 0)
    def _():
        sp_ref[...] = jnp.zeros((1, 1), jnp.float32)

    sp_ref[...] += jnp.sum(jnp.abs(masked)).reshape(1, 1)

    @pl.when(i == nsteps - 1)
    def _():
        sp_ref[...] = sp_ref[...] / denom


def _stageB_body(mc_ref, d_ref, basis_ref, bias_ref, g_ref, b_ref,
                 wq_ref, wk_ref, wv_ref,
                 q_ref, k_ref, v_ref, e_ref, *, nsteps, denom):
    i = pl.program_id(0)
    xr = (basis_ref[...] + bias_ref[...]
          + jnp.dot(mc_ref[...], d_ref[...]))

    @pl.when(i == 0)
    def _():
        e_ref[...] = jnp.zeros((1, 1), jnp.float32)

    e_ref[...] += jnp.sum(xr * xr).reshape(1, 1)

    @pl.when(i == nsteps - 1)
    def _():
        e_ref[...] = e_ref[...] / denom

    n = _ln(xr, g_ref[...], b_ref[...])
    q_ref[...] = jnp.dot(n, wq_ref[...])
    k_ref[...] = jnp.dot(n, wk_ref[...])
    v_ref[...] = jnp.dot(n, wv_ref[...])


def _attn_body(q_ref, k_ref, v_ref, w_ref, ctx_ref, kst_ref, vst_ref,
               *, scale, dh, t_len):
    i = pl.program_id(2)

    @pl.when(i == 0)
    def _():
        k2 = k_ref[0]
        v2 = v_ref[0]
        z = jnp.zeros((t_len, dh), jnp.float32)
        kst_ref[...] = jnp.concatenate([
            jnp.concatenate([k2[:, :dh], z], axis=1),
            jnp.concatenate([z, k2[:, dh:]], axis=1)], axis=0)
        vst_ref[...] = jnp.concatenate([
            jnp.concatenate([v2[:, :dh], z], axis=1),
            jnp.concatenate([z, v2[:, dh:]], axis=1)], axis=0)

    q2 = q_ref[0]
    s_cat = jax.lax.dot_general(q2, kst_ref[...], (((1,), (1,)), ((), ()))) * scale
    w0 = _softmax_last(s_cat[:, :t_len])
    w1 = _softmax_last(s_cat[:, t_len:])
    w_ref[0, 0] = w0
    w_ref[0, 1] = w1
    w_cat = jnp.concatenate([w0, w1], axis=1)
    ctx_ref[0] = jnp.dot(w_cat, vst_ref[...])


def _stageF_body(c_ref, wo_ref, x_ref, g_ref, b_ref, wg_ref, w1_ref, w2_ref,
                 xo_ref, e_ref, *, nsteps, denom):
    f32 = jnp.float32
    bf16 = jnp.bfloat16
    x1 = jnp.dot(c_ref[...].astype(bf16), wo_ref[...],
                 preferred_element_type=f32) + x_ref[...]
    i = pl.program_id(0)

    @pl.when(i == 0)
    def _():
        e_ref[...] = jnp.zeros((1, 1), f32)

    e_ref[...] += jnp.sum(x1 * x1).reshape(1, 1)

    @pl.when(i == nsteps - 1)
    def _():
        e_ref[...] = e_ref[...] / denom

    nb = _ln(x1, g_ref[...], b_ref[...]).astype(bf16)
    h = (jax.nn.sigmoid(jnp.dot(nb, wg_ref[...], preferred_element_type=f32))
         * jax.nn.gelu(jnp.dot(nb, w1_ref[...], preferred_element_type=f32)))
    xo_ref[...] = x1 + jnp.dot(h.astype(bf16), w2_ref[...],
                               preferred_element_type=f32)


def kernel(x, W_fam, fam_emb, W_enc, b_enc, dictionary, bias_correction,
           ln1_g, ln1_b, Wq, Wk, Wv, Wo, ln2_g, ln2_b, W1, Wg, W2):
    B, T, D = x.shape
    F = W_fam.shape[1]
    M = W_enc.shape[1]
    DFF = W1.shape[1]
    N = B * T
    H = _H
    dh = D // H

    R = 256
    nt = N // R
    RA = 512
    nta = N // RA
    TQ = 1024
    f32 = jnp.float32
    bf16 = jnp.bfloat16

    xf = x.reshape(N, D)

    fs, basis, masked, sp = pl.pallas_call(
        functools.partial(_stageA_body, nsteps=nta, denom=float(N * M), n_iter=18),
        grid=(nta,),
        in_specs=[
            pl.BlockSpec((RA, D), lambda i: (i, 0)),
            pl.BlockSpec((D, F), lambda i: (0, 0)),
            pl.BlockSpec((F, D), lambda i: (0, 0)),
            pl.BlockSpec((D, M), lambda i: (0, 0)),
            pl.BlockSpec((1, M), lambda i: (0, 0)),
        ],
        out_specs=[
            pl.BlockSpec((RA, F), lambda i: (i, 0)),
            pl.BlockSpec((RA, D), lambda i: (i, 0)),
            pl.BlockSpec((RA, M), lambda i: (i, 0)),
            pl.BlockSpec((1, 1), lambda i: (0, 0)),
        ],
        out_shape=[
            jax.ShapeDtypeStruct((N, F), f32),
            jax.ShapeDtypeStruct((N, D), f32),
            jax.ShapeDtypeStruct((N, M), f32),
            jax.ShapeDtypeStruct((1, 1), f32),
        ],
    )(xf, W_fam, fam_emb, W_enc, b_enc.reshape(1, M))

    qf, kf, vf, e1 = pl.pallas_call(
        functools.partial(_stageB_body, nsteps=nt, denom=float(N * D)),
        grid=(nt,),
        in_specs=[
            pl.BlockSpec((R, M), lambda i: (i, 0)),
            pl.BlockSpec((M, D), lambda i: (0, 0)),
            pl.BlockSpec((R, D), lambda i: (i, 0)),
            pl.BlockSpec((1, D), lambda i: (0, 0)),
            pl.BlockSpec((1, D), lambda i: (0, 0)),
            pl.BlockSpec((1, D), lambda i: (0, 0)),
            pl.BlockSpec((D, D), lambda i: (0, 0)),
            pl.BlockSpec((D, D), lambda i: (0, 0)),
            pl.BlockSpec((D, D), lambda i: (0, 0)),
        ],
        out_specs=[
            pl.BlockSpec((R, D), lambda i: (i, 0)),
            pl.BlockSpec((R, D), lambda i: (i, 0)),
            pl.BlockSpec((R, D), lambda i: (i, 0)),
            pl.BlockSpec((1, 1), lambda i: (0, 0)),
        ],
        out_shape=[
            jax.ShapeDtypeStruct((N, D), f32),
            jax.ShapeDtypeStruct((N, D), f32),
            jax.ShapeDtypeStruct((N, D), f32),
            jax.ShapeDtypeStruct((1, 1), f32),
        ],
    )(masked, dictionary, basis, bias_correction.reshape(1, D),
      ln1_g.reshape(1, D), ln1_b.reshape(1, D), Wq, Wk, Wv)

    q3 = qf.reshape(B, T, D)
    k3 = kf.reshape(B, T, D)
    v3 = vf.reshape(B, T, D)

    attn_w, ctx3 = pl.pallas_call(
        functools.partial(_attn_body, scale=1.0 / float(dh) ** 0.5, dh=dh, t_len=T),
        grid=(B, H // 2, T // TQ),
        in_specs=[
            pl.BlockSpec((1, TQ, 2 * dh), lambda b, hp, i: (b, i, hp)),
            pl.BlockSpec((1, T, 2 * dh), lambda b, hp, i: (b, 0, hp)),
            pl.BlockSpec((1, T, 2 * dh), lambda b, hp, i: (b, 0, hp)),
        ],
        out_specs=[
            pl.BlockSpec((1, 2, TQ, T), lambda b, hp, i: (b, hp, i, 0)),
            pl.BlockSpec((1, TQ, 2 * dh), lambda b, hp, i: (b, i, hp)),
        ],
        out_shape=[
            jax.ShapeDtypeStruct((B, H, T, T), f32),
            jax.ShapeDtypeStruct((B, T, D), f32),
        ],
        scratch_shapes=[
            pltpu.VMEM((2 * T, 2 * dh), f32),
            pltpu.VMEM((2 * T, 2 * dh), f32),
        ],
    )(q3, k3, v3)

    ctx = ctx3.reshape(N, D)

    xout, e2 = pl.pallas_call(
        functools.partial(_stageF_body, nsteps=nta, denom=float(N * D)),
        grid=(nta,),
        in_specs=[
            pl.BlockSpec((RA, D), lambda i: (i, 0)),
            pl.BlockSpec((D, D), lambda i: (0, 0)),
            pl.BlockSpec((RA, D), lambda i: (i, 0)),
            pl.BlockSpec((1, D), lambda i: (0, 0)),
            pl.BlockSpec((1, D), lambda i: (0, 0)),
            pl.BlockSpec((D, DFF), lambda i: (0, 0)),
            pl.BlockSpec((D, DFF), lambda i: (0, 0)),
            pl.BlockSpec((DFF, D), lambda i: (0, 0)),
        ],
        out_specs=[
            pl.BlockSpec((RA, D), lambda i: (i, 0)),
            pl.BlockSpec((1, 1), lambda i: (0, 0)),
        ],
        out_shape=[
            jax.ShapeDtypeStruct((N, D), f32),
            jax.ShapeDtypeStruct((1, 1), f32),
        ],
    )(ctx, Wo.astype(bf16), xf, ln2_g.reshape(1, D), ln2_b.reshape(1, D),
      Wg.astype(bf16), W1.astype(bf16), W2.astype(bf16))

    return (
        xout.reshape(B, T, D),
        attn_w,
        fs.reshape(B, T, F),
        sp.reshape(()),
        e1.reshape(()),
        e2.reshape(()),
    )


# speedup vs baseline: 1.0762x; 1.0182x over previous
import functools

import jax
import jax.numpy as jnp
from jax.experimental import pallas as pl
from jax.experimental.pallas import tpu as pltpu

_K = 64
_H = 16
_LN_EPS = 1e-5


def _ln(xx, g, b):
    mu = jnp.mean(xx, axis=-1, keepdims=True)
    var = jnp.mean((xx - mu) ** 2, axis=-1, keepdims=True)
    return (xx - mu) / jnp.sqrt(var + _LN_EPS) * g + b


def _softmax_last(s):
    s = s - jnp.max(s, axis=-1, keepdims=True)
    e = jnp.exp(s)
    return e / jnp.sum(e, axis=-1, keepdims=True)


def _stageA_body(x_ref, wfam_ref, femb_ref, wenc_ref, benc_ref,
                 fs_ref, basis_ref, mc_ref, sp_ref, *, nsteps, denom, n_iter):
    xx = x_ref[...]
    s = jnp.dot(xx, wfam_ref[...])
    s = s - jnp.max(s, axis=-1, keepdims=True)
    e = jnp.exp(s)
    p = e / jnp.sum(e, axis=-1, keepdims=True)
    fs_ref[...] = p
    basis = jnp.dot(p, femb_ref[...])
    basis_ref[...] = basis
    c = jnp.dot(xx - basis, wenc_ref[...]) + benc_ref[...]

    mu = jnp.mean(c, axis=1, keepdims=True)
    sg = jnp.sqrt(jnp.maximum(
        jnp.mean(c * c, axis=1, keepdims=True) - mu * mu, 0.0))
    lo0 = mu + sg * 1.7
    hi0 = mu + sg * 2.6

    def _cnt(t):
        return jnp.sum((c >= t).astype(jnp.float32), axis=1, keepdims=True)

    lo = jnp.where(_cnt(lo0) >= _K, lo0, mu - sg * 9.0)
    hi = jnp.where(_cnt(hi0) < _K, hi0, mu + sg * 9.0)

    def it(_, lh):
        lo_, hi_ = lh
        mid = (lo_ + hi_) * 0.5
        ge = _cnt(mid) >= _K
        return (jnp.where(ge, mid, lo_), jnp.where(ge, hi_, mid))

    lo, hi = jax.lax.fori_loop(0, n_iter, it, (lo, hi))
    masked = jnp.where(c >= lo, c, 0.0)
    mc_ref[...] = masked

    i = pl.program_id(0)

    @pl.when(i == 0)
    def _():
        sp_ref[...] = jnp.zeros((1, 1), jnp.float32)

    sp_ref[...] += jnp.sum(jnp.abs(masked)).reshape(1, 1)

    @pl.when(i == nsteps - 1)
    def _():
        sp_ref[...] = sp_ref[...] / denom


def _stageB_body(mc_ref, d_ref, basis_ref, bias_ref, g_ref, b_ref,
                 wq_ref, wk_ref, wv_ref,
                 q_ref, k_ref, v_ref, e_ref, *, nsteps, denom):
    i = pl.program_id(0)
    xr = (basis_ref[...] + bias_ref[...]
          + jnp.dot(mc_ref[...], d_ref[...]))

    @pl.when(i == 0)
    def _():
        e_ref[...] = jnp.zeros((1, 1), jnp.float32)

    e_ref[...] += jnp.sum(xr * xr).reshape(1, 1)

    @pl.when(i == nsteps - 1)
    def _():
        e_ref[...] = e_ref[...] / denom

    n = _ln(xr, g_ref[...], b_ref[...])
    q_ref[...] = jnp.dot(n, wq_ref[...])
    k_ref[...] = jnp.dot(n, wk_ref[...])
    v_ref[...] = jnp.dot(n, wv_ref[...])


def _attn_body(q_ref, k_ref, v_ref, w_ref, ctx_ref, kst_ref, vst_ref,
               *, scale, dh, t_len):
    i = pl.program_id(2)

    @pl.when(i == 0)
    def _():
        k2 = k_ref[0]
        v2 = v_ref[0]
        z = jnp.zeros((t_len, dh), jnp.float32)
        kst_ref[...] = jnp.concatenate([
            jnp.concatenate([k2[:, :dh], z], axis=1),
            jnp.concatenate([z, k2[:, dh:]], axis=1)], axis=0)
        vst_ref[...] = jnp.concatenate([
            jnp.concatenate([v2[:, :dh], z], axis=1),
            jnp.concatenate([z, v2[:, dh:]], axis=1)], axis=0)

    q2 = q_ref[0]
    s_cat = jax.lax.dot_general(q2, kst_ref[...], (((1,), (1,)), ((), ()))) * scale
    w0 = _softmax_last(s_cat[:, :t_len])
    w1 = _softmax_last(s_cat[:, t_len:])
    w_ref[0, 0] = w0
    w_ref[0, 1] = w1
    w_cat = jnp.concatenate([w0, w1], axis=1)
    ctx_ref[0] = jnp.dot(w_cat, vst_ref[...])


def _stageF_body(c_ref, wo_ref, x_ref, g_ref, b_ref, wg_ref, w1_ref, w2_ref,
                 xo_ref, e_ref, *, nsteps, denom):
    f32 = jnp.float32
    bf16 = jnp.bfloat16
    x1 = jnp.dot(c_ref[...].astype(bf16), wo_ref[...],
                 preferred_element_type=f32) + x_ref[...]
    i = pl.program_id(0)

    @pl.when(i == 0)
    def _():
        e_ref[...] = jnp.zeros((1, 1), f32)

    e_ref[...] += jnp.sum(x1 * x1).reshape(1, 1)

    @pl.when(i == nsteps - 1)
    def _():
        e_ref[...] = e_ref[...] / denom

    nb = _ln(x1, g_ref[...], b_ref[...]).astype(bf16)
    h = (jax.nn.sigmoid(jnp.dot(nb, wg_ref[...], preferred_element_type=f32))
         * jax.nn.gelu(jnp.dot(nb, w1_ref[...], preferred_element_type=f32)))
    xo_ref[...] = x1 + jnp.dot(h.astype(bf16), w2_ref[...],
                               preferred_element_type=f32)


def kernel(x, W_fam, fam_emb, W_enc, b_enc, dictionary, bias_correction,
           ln1_g, ln1_b, Wq, Wk, Wv, Wo, ln2_g, ln2_b, W1, Wg, W2):
    B, T, D = x.shape
    F = W_fam.shape[1]
    M = W_enc.shape[1]
    DFF = W1.shape[1]
    N = B * T
    H = _H
    dh = D // H

    R = 256
    nt = N // R
    RA = 512
    nta = N // RA
    TQ = 1024
    f32 = jnp.float32
    bf16 = jnp.bfloat16

    xf = x.reshape(N, D)

    fs, basis, masked, sp = pl.pallas_call(
        functools.partial(_stageA_body, nsteps=nta, denom=float(N * M), n_iter=16),
        grid=(nta,),
        in_specs=[
            pl.BlockSpec((RA, D), lambda i: (i, 0)),
            pl.BlockSpec((D, F), lambda i: (0, 0)),
            pl.BlockSpec((F, D), lambda i: (0, 0)),
            pl.BlockSpec((D, M), lambda i: (0, 0)),
            pl.BlockSpec((1, M), lambda i: (0, 0)),
        ],
        out_specs=[
            pl.BlockSpec((RA, F), lambda i: (i, 0)),
            pl.BlockSpec((RA, D), lambda i: (i, 0)),
            pl.BlockSpec((RA, M), lambda i: (i, 0)),
            pl.BlockSpec((1, 1), lambda i: (0, 0)),
        ],
        out_shape=[
            jax.ShapeDtypeStruct((N, F), f32),
            jax.ShapeDtypeStruct((N, D), f32),
            jax.ShapeDtypeStruct((N, M), f32),
            jax.ShapeDtypeStruct((1, 1), f32),
        ],
    )(xf, W_fam, fam_emb, W_enc, b_enc.reshape(1, M))

    qf, kf, vf, e1 = pl.pallas_call(
        functools.partial(_stageB_body, nsteps=nt, denom=float(N * D)),
        grid=(nt,),
        in_specs=[
            pl.BlockSpec((R, M), lambda i: (i, 0)),
            pl.BlockSpec((M, D), lambda i: (0, 0)),
            pl.BlockSpec((R, D), lambda i: (i, 0)),
            pl.BlockSpec((1, D), lambda i: (0, 0)),
            pl.BlockSpec((1, D), lambda i: (0, 0)),
            pl.BlockSpec((1, D), lambda i: (0, 0)),
            pl.BlockSpec((D, D), lambda i: (0, 0)),
            pl.BlockSpec((D, D), lambda i: (0, 0)),
            pl.BlockSpec((D, D), lambda i: (0, 0)),
        ],
        out_specs=[
            pl.BlockSpec((R, D), lambda i: (i, 0)),
            pl.BlockSpec((R, D), lambda i: (i, 0)),
            pl.BlockSpec((R, D), lambda i: (i, 0)),
            pl.BlockSpec((1, 1), lambda i: (0, 0)),
        ],
        out_shape=[
            jax.ShapeDtypeStruct((N, D), f32),
            jax.ShapeDtypeStruct((N, D), f32),
            jax.ShapeDtypeStruct((N, D), f32),
            jax.ShapeDtypeStruct((1, 1), f32),
        ],
    )(masked, dictionary, basis, bias_correction.reshape(1, D),
      ln1_g.reshape(1, D), ln1_b.reshape(1, D), Wq, Wk, Wv)

    q3 = qf.reshape(B, T, D)
    k3 = kf.reshape(B, T, D)
    v3 = vf.reshape(B, T, D)

    attn_w, ctx3 = pl.pallas_call(
        functools.partial(_attn_body, scale=1.0 / float(dh) ** 0.5, dh=dh, t_len=T),
        grid=(B, H // 2, T // TQ),
        in_specs=[
            pl.BlockSpec((1, TQ, 2 * dh), lambda b, hp, i: (b, i, hp)),
            pl.BlockSpec((1, T, 2 * dh), lambda b, hp, i: (b, 0, hp)),
            pl.BlockSpec((1, T, 2 * dh), lambda b, hp, i: (b, 0, hp)),
        ],
        out_specs=[
            pl.BlockSpec((1, 2, TQ, T), lambda b, hp, i: (b, hp, i, 0)),
            pl.BlockSpec((1, TQ, 2 * dh), lambda b, hp, i: (b, i, hp)),
        ],
        out_shape=[
            jax.ShapeDtypeStruct((B, H, T, T), f32),
            jax.ShapeDtypeStruct((B, T, D), f32),
        ],
        scratch_shapes=[
            pltpu.VMEM((2 * T, 2 * dh), f32),
            pltpu.VMEM((2 * T, 2 * dh), f32),
        ],
    )(q3, k3, v3)

    ctx = ctx3.reshape(N, D)

    xout, e2 = pl.pallas_call(
        functools.partial(_stageF_body, nsteps=nta, denom=float(N * D)),
        grid=(nta,),
        in_specs=[
            pl.BlockSpec((RA, D), lambda i: (i, 0)),
            pl.BlockSpec((D, D), lambda i: (0, 0)),
            pl.BlockSpec((RA, D), lambda i: (i, 0)),
            pl.BlockSpec((1, D), lambda i: (0, 0)),
            pl.BlockSpec((1, D), lambda i: (0, 0)),
            pl.BlockSpec((D, DFF), lambda i: (0, 0)),
            pl.BlockSpec((D, DFF), lambda i: (0, 0)),
            pl.BlockSpec((DFF, D), lambda i: (0, 0)),
        ],
        out_specs=[
            pl.BlockSpec((RA, D), lambda i: (i, 0)),
            pl.BlockSpec((1, 1), lambda i: (0, 0)),
        ],
        out_shape=[
            jax.ShapeDtypeStruct((N, D), f32),
            jax.ShapeDtypeStruct((1, 1), f32),
        ],
    )(ctx, Wo.astype(bf16), xf, ln2_g.reshape(1, D), ln2_b.reshape(1, D),
      Wg.astype(bf16), W1.astype(bf16), W2.astype(bf16))

    return (
        xout.reshape(B, T, D),
        attn_w,
        fs.reshape(B, T, F),
        sp.reshape(()),
        e1.reshape(()),
        e2.reshape(()),
    )
